# baseline (speedup 1.0000x reference)
import functools

import jax
import jax.numpy as jnp
from jax import lax
from jax.experimental import pallas as pl
from jax.experimental.pallas import tpu as pltpu
from jax.experimental.pallas import tpu_sc as plsc

MARGIN_C = 0.05
N_MEANS = 65536
N_BATCH = 16
N_PTS = 1024
N_Q = N_BATCH * N_PTS
KT = 65536
W = 128
QC = 256
BIG = 3.0e38
IDX_BIG = 1.6e7


def _tc_kernel(p_ref, a_ref, m_ref, idx_ref, q_ref, rund_ref, runi_ref):
    t = pl.program_id(1)
    n_kt = pl.num_programs(1)

    pts = p_ref[0]
    A = a_ref[0]
    qfull = jnp.dot(pts.astype(jnp.bfloat16), A.astype(jnp.bfloat16),
                    preferred_element_type=jnp.float32)
    q8 = qfull[:, 0:8] + A[3:4, 0:8]
    qb8 = q8.astype(jnp.bfloat16)

    @pl.when(t == 0)
    def _():
        q_ref[0] = q8
        rund_ref[...] = jnp.full((N_PTS, W), BIG, jnp.float32)
        runi_ref[...] = jnp.zeros((N_PTS, W), jnp.float32)

    iota_l = lax.broadcasted_iota(jnp.int32, (1, W), 1).astype(jnp.float32)
    base_f = lax.convert_element_type(t * (KT // W) * W, jnp.float32)

    for qc in range(N_PTS // QC):
        sl = pl.ds(qc * QC, QC)
        qbq = qb8[qc * QC:(qc + 1) * QC, :]
        rund = rund_ref[sl, :]
        runi = runi_ref[sl, :]
        for u in range(KT // W):
            m3 = m_ref[u]
            dot = jnp.dot(qbq, m3.astype(jnp.bfloat16),
                          preferred_element_type=jnp.float32)
            tv = m3[3:4, :] + dot
            idx_c = iota_l + (base_f + float(u * W))
            take = tv < rund
            rund = jnp.where(take, tv, rund)
            runi = jnp.where(take, idx_c, runi)
        rund_ref[sl, :] = rund
        runi_ref[sl, :] = runi

        @pl.when(t == n_kt - 1)
        def _():
            mind = jnp.min(rund, axis=1, keepdims=True)
            cand = jnp.where(rund == mind, runi, IDX_BIG)
            idxq = jnp.min(cand, axis=1, keepdims=True)
            idx_ref[0, sl, :] = jnp.broadcast_to(idxq, (QC, 8))


def _sc_stage(mx_h, my_h, mz_h, idx_i32, qx, qy, qz):
    info = plsc.get_sparse_core_info()
    nc, ns = info.num_cores, info.num_subcores
    nw = nc * ns
    bpw = N_Q // nw
    mesh = plsc.VectorSubcoreMesh(core_axis_name="c", subcore_axis_name="s")

    @functools.partial(
        pl.kernel, mesh=mesh,
        out_type=jax.ShapeDtypeStruct((nw, 16), jnp.float32),
        scratch_types=[
            pltpu.VMEM((bpw,), jnp.int32),
            pltpu.VMEM((bpw,), jnp.float32),
            pltpu.VMEM((bpw,), jnp.float32),
            pltpu.VMEM((bpw,), jnp.float32),
            pltpu.VMEM((bpw,), jnp.float32),
            pltpu.VMEM((bpw,), jnp.float32),
            pltpu.VMEM((bpw,), jnp.float32),
            pltpu.VMEM((16,), jnp.float32),
            pltpu.SemaphoreType.DMA,
            pltpu.SemaphoreType.DMA,
            pltpu.SemaphoreType.DMA,
        ],
    )
    def sc_body(mx_hbm, my_hbm, mz_hbm, idx_hbm, qx_hbm, qy_hbm, qz_hbm,
                out_hbm, idx_v, mx_v, my_v, mz_v, qx_v, qy_v, qz_v, acc_v,
                sem0, sem1, sem2):
        wid = lax.axis_index("s") * nc + lax.axis_index("c")
        base = wid * bpw
        pltpu.sync_copy(idx_hbm.at[pl.ds(base, bpw)], idx_v)
        cx = pltpu.async_copy(mx_hbm.at[idx_v], mx_v, sem0)
        cy = pltpu.async_copy(my_hbm.at[idx_v], my_v, sem1)
        cz = pltpu.async_copy(mz_hbm.at[idx_v], mz_v, sem2)
        pltpu.sync_copy(qx_hbm.at[pl.ds(base, bpw)], qx_v)
        pltpu.sync_copy(qy_hbm.at[pl.ds(base, bpw)], qy_v)
        pltpu.sync_copy(qz_hbm.at[pl.ds(base, bpw)], qz_v)
        cx.wait()
        cy.wait()
        cz.wait()

        acc = jnp.zeros((16,), jnp.float32)
        for i in range(bpw // 16):
            sl = pl.ds(i * 16, 16)
            dx = qx_v[sl] - mx_v[sl]
            dy = qy_v[sl] - my_v[sl]
            dz = qz_v[sl] - mz_v[sl]
            d2 = (dx * dx + dy * dy) + dz * dz
            acc = acc + jnp.maximum(MARGIN_C - d2, 0.0)
        acc_v[...] = acc
        pltpu.sync_copy(acc_v, out_hbm.at[wid])

    return sc_body(mx_h, my_h, mz_h, idx_i32, qx, qy, qz)


@jax.jit
def kernel(outputs, c2ws, scene_scales, means):
    aff = c2ws[:, :3, :3] * scene_scales[:, None, None]
    trans = c2ws[:, :3, 3]
    pts8 = jnp.pad(outputs, ((0, 0), (0, 0), (0, 5)))
    affT = jnp.transpose(aff, (0, 2, 1))
    abar = jnp.zeros((N_BATCH, 8, 128), jnp.float32)
    abar = abar.at[:, 0:3, 0:3].set(affT)
    abar = abar.at[:, 3, 0:3].set(trans)
    msq = jnp.sum(means * means, axis=1)
    mt = jnp.concatenate(
        [-2.0 * means.T, msq[None, :], jnp.zeros((4, N_MEANS), jnp.float32)],
        axis=0)
    mt3 = jnp.transpose(
        mt.reshape(8, N_MEANS // W, W), (1, 0, 2))

    idxs, qs = pl.pallas_call(
        _tc_kernel,
        grid=(N_BATCH, N_MEANS // KT),
        in_specs=[
            pl.BlockSpec((1, N_PTS, 8), lambda b, t: (b, 0, 0)),
            pl.BlockSpec((1, 8, 128), lambda b, t: (b, 0, 0)),
            pl.BlockSpec((KT // W, 8, W), lambda b, t: (t, 0, 0)),
        ],
        out_specs=[
            pl.BlockSpec((1, N_PTS, 8), lambda b, t: (b, 0, 0)),
            pl.BlockSpec((1, N_PTS, 8), lambda b, t: (b, 0, 0)),
        ],
        out_shape=[
            jax.ShapeDtypeStruct((N_BATCH, N_PTS, 8), jnp.float32),
            jax.ShapeDtypeStruct((N_BATCH, N_PTS, 8), jnp.float32),
        ],
        scratch_shapes=[
            pltpu.VMEM((N_PTS, W), jnp.float32),
            pltpu.VMEM((N_PTS, W), jnp.float32),
        ],
        compiler_params=pltpu.CompilerParams(
            dimension_semantics=("parallel", "arbitrary")),
    )(pts8, abar, mt3)

    idx_i32 = idxs[:, :, 0].reshape(N_Q).astype(jnp.int32)
    qx = qs[:, :, 0].reshape(N_Q)
    qy = qs[:, :, 1].reshape(N_Q)
    qz = qs[:, :, 2].reshape(N_Q)
    mx_h = means[:, 0]
    my_h = means[:, 1]
    mz_h = means[:, 2]

    partials = _sc_stage(mx_h, my_h, mz_h, idx_i32, qx, qy, qz)
    return jnp.sum(partials) / float(N_Q)


# speedup vs baseline: 1.3771x; 1.3771x over previous
import functools

import jax
import jax.numpy as jnp
from jax import lax
from jax.experimental import pallas as pl
from jax.experimental.pallas import tpu as pltpu
from jax.experimental.pallas import tpu_sc as plsc

MARGIN_C = 0.05
N_MEANS = 65536
N_BATCH = 16
N_PTS = 1024
N_Q = N_BATCH * N_PTS
KT = 32768
W = 128
QC = 256
BIG = 3.0e38
IDX_BIG = 1.6e7


def _tc_kernel(p_ref, a_ref, m_ref, idx_ref, q_ref, rund_ref, runi_ref):
    t = pl.program_id(1)
    n_kt = pl.num_programs(1)

    pts = p_ref[0]
    A = a_ref[0]
    qfull = jnp.dot(pts.astype(jnp.bfloat16), A.astype(jnp.bfloat16),
                    preferred_element_type=jnp.float32)
    q8 = qfull[:, 0:8] + A[3:4, 0:8]
    qb8 = q8.astype(jnp.bfloat16)

    @pl.when(t == 0)
    def _():
        q_ref[0] = q8
        rund_ref[...] = jnp.full((N_PTS, W), BIG, jnp.float32)
        runi_ref[...] = jnp.zeros((N_PTS, W), jnp.float32)

    iota_l = lax.broadcasted_iota(jnp.int32, (1, W), 1).astype(jnp.float32)
    base_f = lax.convert_element_type(t * (KT // W) * W, jnp.float32)

    for qc in range(N_PTS // QC):
        sl = pl.ds(qc * QC, QC)
        qbq = qb8[qc * QC:(qc + 1) * QC, :]
        rund = rund_ref[sl, :]
        runi = runi_ref[sl, :]
        for u in range(KT // W):
            m3 = m_ref[u]
            dot = jnp.dot(qbq, m3.astype(jnp.bfloat16),
                          preferred_element_type=jnp.float32)
            tv = m3[3:4, :] + dot
            idx_c = iota_l + (base_f + float(u * W))
            take = tv < rund
            rund = jnp.where(take, tv, rund)
            runi = jnp.where(take, idx_c, runi)
        rund_ref[sl, :] = rund
        runi_ref[sl, :] = runi

        @pl.when(t == n_kt - 1)
        def _():
            mind = jnp.min(rund, axis=1, keepdims=True)
            cand = jnp.where(rund == mind, runi, IDX_BIG)
            idxq = jnp.min(cand, axis=1, keepdims=True)
            idx_ref[0, sl, :] = jnp.broadcast_to(idxq, (QC, 8))


def _sc_stage(mx_h, my_h, mz_h, idx_i32, qx, qy, qz):
    info = plsc.get_sparse_core_info()
    nc, ns = info.num_cores, info.num_subcores
    nw = nc * ns
    bpw = N_Q // nw
    mesh = plsc.VectorSubcoreMesh(core_axis_name="c", subcore_axis_name="s")

    @functools.partial(
        pl.kernel, mesh=mesh,
        out_type=jax.ShapeDtypeStruct((nw, 16), jnp.float32),
        scratch_types=[
            pltpu.VMEM((bpw,), jnp.int32),
            pltpu.VMEM((bpw,), jnp.float32),
            pltpu.VMEM((bpw,), jnp.float32),
            pltpu.VMEM((bpw,), jnp.float32),
            pltpu.VMEM((bpw,), jnp.float32),
            pltpu.VMEM((bpw,), jnp.float32),
            pltpu.VMEM((bpw,), jnp.float32),
            pltpu.VMEM((16,), jnp.float32),
            pltpu.SemaphoreType.DMA,
            pltpu.SemaphoreType.DMA,
            pltpu.SemaphoreType.DMA,
        ],
    )
    def sc_body(mx_hbm, my_hbm, mz_hbm, idx_hbm, qx_hbm, qy_hbm, qz_hbm,
                out_hbm, idx_v, mx_v, my_v, mz_v, qx_v, qy_v, qz_v, acc_v,
                sem0, sem1, sem2):
        wid = lax.axis_index("s") * nc + lax.axis_index("c")
        base = wid * bpw
        pltpu.sync_copy(idx_hbm.at[pl.ds(base, bpw)], idx_v)
        cx = pltpu.async_copy(mx_hbm.at[idx_v], mx_v, sem0)
        cy = pltpu.async_copy(my_hbm.at[idx_v], my_v, sem1)
        cz = pltpu.async_copy(mz_hbm.at[idx_v], mz_v, sem2)
        pltpu.sync_copy(qx_hbm.at[pl.ds(base, bpw)], qx_v)
        pltpu.sync_copy(qy_hbm.at[pl.ds(base, bpw)], qy_v)
        pltpu.sync_copy(qz_hbm.at[pl.ds(base, bpw)], qz_v)
        cx.wait()
        cy.wait()
        cz.wait()

        acc = jnp.zeros((16,), jnp.float32)
        for i in range(bpw // 16):
            sl = pl.ds(i * 16, 16)
            dx = qx_v[sl] - mx_v[sl]
            dy = qy_v[sl] - my_v[sl]
            dz = qz_v[sl] - mz_v[sl]
            d2 = (dx * dx + dy * dy) + dz * dz
            acc = acc + jnp.maximum(MARGIN_C - d2, 0.0)
        acc_v[...] = acc
        pltpu.sync_copy(acc_v, out_hbm.at[wid])

    return sc_body(mx_h, my_h, mz_h, idx_i32, qx, qy, qz)


@jax.jit
def kernel(outputs, c2ws, scene_scales, means):
    aff = c2ws[:, :3, :3] * scene_scales[:, None, None]
    trans = c2ws[:, :3, 3]
    pts8 = jnp.pad(outputs, ((0, 0), (0, 0), (0, 5)))
    affT = jnp.transpose(aff, (0, 2, 1))
    abar = jnp.zeros((N_BATCH, 8, 128), jnp.float32)
    abar = abar.at[:, 0:3, 0:3].set(affT)
    abar = abar.at[:, 3, 0:3].set(trans)
    msq = jnp.sum(means * means, axis=1)
    mt = jnp.concatenate(
        [-2.0 * means.T, msq[None, :], jnp.zeros((4, N_MEANS), jnp.float32)],
        axis=0)
    mt3 = jnp.transpose(
        mt.reshape(8, N_MEANS // W, W), (1, 0, 2))

    idxs, qs = pl.pallas_call(
        _tc_kernel,
        grid=(N_BATCH, N_MEANS // KT),
        in_specs=[
            pl.BlockSpec((1, N_PTS, 8), lambda b, t: (b, 0, 0)),
            pl.BlockSpec((1, 8, 128), lambda b, t: (b, 0, 0)),
            pl.BlockSpec((KT // W, 8, W), lambda b, t: (t, 0, 0)),
        ],
        out_specs=[
            pl.BlockSpec((1, N_PTS, 8), lambda b, t: (b, 0, 0)),
            pl.BlockSpec((1, N_PTS, 8), lambda b, t: (b, 0, 0)),
        ],
        out_shape=[
            jax.ShapeDtypeStruct((N_BATCH, N_PTS, 8), jnp.float32),
            jax.ShapeDtypeStruct((N_BATCH, N_PTS, 8), jnp.float32),
        ],
        scratch_shapes=[
            pltpu.VMEM((N_PTS, W), jnp.float32),
            pltpu.VMEM((N_PTS, W), jnp.float32),
        ],
        compiler_params=pltpu.CompilerParams(
            dimension_semantics=("parallel", "arbitrary")),
    )(pts8, abar, mt3)

    idx_i32 = idxs[:, :, 0].reshape(N_Q).astype(jnp.int32)
    qx = qs[:, :, 0].reshape(N_Q)
    qy = qs[:, :, 1].reshape(N_Q)
    qz = qs[:, :, 2].reshape(N_Q)
    mx_h = means[:, 0]
    my_h = means[:, 1]
    mz_h = means[:, 2]

    partials = _sc_stage(mx_h, my_h, mz_h, idx_i32, qx, qy, qz)
    return jnp.sum(partials) / float(N_Q)
